# baseline (speedup 1.0000x reference)
import functools

import jax
import jax.numpy as jnp
from jax import lax
from jax.experimental import pallas as pl
from jax.experimental.pallas import tpu as pltpu
from jax.experimental.pallas import tpu_sc as plsc

NC = 2
NS = 16
LANES = 16
CH = 128
NH = 2


def _sc_agg_body(nrel_per_core, cpt_h, rows_per_tile, d,
                 emb_hbm, src_hbm, dst_hbm, w_hbm, out_hbm,
                 src_v, dst_v, w_v, rows_a, rows_b, acc_sh,
                 sem_a, sem_b, sem_sa, sem_sb):
    c = lax.axis_index("c")
    s = lax.axis_index("s")
    base = s * rows_per_tile
    nd16 = d // LANES
    half = CH // 2
    zeros16 = jnp.zeros((LANES,), jnp.float32)

    def zero_row(i, _):
        for dd in range(nd16):
            rows_a[i, pl.ds(dd * LANES, LANES)] = zeros16
        return 0

    def scale_half(rows, j, hh):
        idx_j = jnp.full((LANES,), j, jnp.int32)

        @plsc.parallel_loop(hh * half, (hh + 1) * half, unroll=8)
        def _(e):
            idx_e = jnp.full((LANES,), e, jnp.int32)
            wb = plsc.load_gather(w_v, [idx_j, idx_e])
            for dd in range(nd16):
                sl = pl.ds(dd * LANES, LANES)
                rows[e, sl] = rows[e, sl] * wb

    def scatter_half(rows, j, hh, sem):
        pltpu.async_copy(rows.at[pl.ds(hh * half, half)],
                         acc_sh.at[dst_v.at[j, pl.ds(hh * half, half)]],
                         sem, add=True)

    def drain_scatters(rows, sem):
        for _ in range(2):
            pltpu.make_async_copy(rows.at[pl.ds(0, half)],
                                  acc_sh.at[dst_v.at[0, pl.ds(0, half)]],
                                  sem).wait()

    for rr in range(nrel_per_core):
        r = c * nrel_per_core + rr

        lax.fori_loop(0, CH, zero_row, 0)
        n_sub, rem = divmod(rows_per_tile, CH)
        for i in range(n_sub):
            pltpu.sync_copy(rows_a.at[pl.ds(0, CH)],
                            acc_sh.at[pl.ds(base + i * CH, CH)])
        if rem:
            pltpu.sync_copy(rows_a.at[pl.ds(0, rem)],
                            acc_sh.at[pl.ds(base + n_sub * CH, rem)])
        plsc.subcore_barrier()

        for h in range(NH):
            pltpu.sync_copy(src_hbm.at[r, s, h], src_v)
            pltpu.sync_copy(dst_hbm.at[r, s, h], dst_v)
            pltpu.sync_copy(w_hbm.at[r, s, h], w_v)

            pltpu.async_copy(emb_hbm.at[src_v.at[0]], rows_a, sem_a)

            def chunk_pair(k, _):
                j0 = 2 * k
                pltpu.make_async_copy(emb_hbm.at[src_v.at[j0]], rows_a,
                                      sem_a).wait()
                scale_half(rows_a, j0, 0)
                scatter_half(rows_a, j0, 0, sem_sa)

                @pl.when(j0 + 1 < cpt_h)
                def _():
                    @pl.when(j0 >= 1)
                    def _():
                        drain_scatters(rows_b, sem_sb)
                    pltpu.async_copy(emb_hbm.at[src_v.at[j0 + 1]], rows_b,
                                     sem_b)
                scale_half(rows_a, j0, 1)
                scatter_half(rows_a, j0, 1, sem_sa)

                @pl.when(j0 + 1 < cpt_h)
                def _():
                    j1 = j0 + 1
                    pltpu.make_async_copy(emb_hbm.at[src_v.at[j1]], rows_b,
                                          sem_b).wait()
                    scale_half(rows_b, j1, 0)
                    scatter_half(rows_b, j1, 0, sem_sb)

                    @pl.when(j1 + 1 < cpt_h)
                    def _():
                        drain_scatters(rows_a, sem_sa)
                        pltpu.async_copy(emb_hbm.at[src_v.at[j1 + 1]], rows_a,
                                         sem_a)
                    scale_half(rows_b, j1, 1)
                    scatter_half(rows_b, j1, 1, sem_sb)
                return 0

            lax.fori_loop(0, (cpt_h + 1) // 2, chunk_pair, 0)
            drain_scatters(rows_a, sem_sa)
            drain_scatters(rows_b, sem_sb)

        plsc.subcore_barrier()
        pltpu.sync_copy(acc_sh.at[pl.ds(base, rows_per_tile)],
                        out_hbm.at[r, pl.ds(base, rows_per_tile)])
        plsc.subcore_barrier()


def _make_sc_agg(n, d, nrel, cpt_h):
    rows_per_tile = n // NS
    mesh = plsc.VectorSubcoreMesh(core_axis_name="c", subcore_axis_name="s")
    body = functools.partial(_sc_agg_body, nrel // NC, cpt_h, rows_per_tile, d)
    return pl.kernel(
        body,
        out_type=jax.ShapeDtypeStruct((nrel, n, d), jnp.float32),
        mesh=mesh,
        compiler_params=pltpu.CompilerParams(needs_layout_passes=False),
        scratch_types=[
            pltpu.VMEM((cpt_h, CH), jnp.int32),
            pltpu.VMEM((cpt_h, CH), jnp.int32),
            pltpu.VMEM((cpt_h, CH), jnp.float32),
            pltpu.VMEM((CH, d), jnp.float32),
            pltpu.VMEM((CH, d), jnp.float32),
            pltpu.VMEM_SHARED((n, d), jnp.float32),
            pltpu.SemaphoreType.DMA,
            pltpu.SemaphoreType.DMA,
            pltpu.SemaphoreType.DMA,
            pltpu.SemaphoreType.DMA,
        ],
    )


def _mm_body(x_ref, w_ref, o_ref):
    o_ref[...] = jnp.dot(x_ref[...], w_ref[...],
                         preferred_element_type=jnp.float32)


def _transform_body(nrel, normalize, acc_ref, w_ref, o_ref):
    out = lax.dot_general(acc_ref[0], w_ref[0], (((1,), (1,)), ((), ())),
                          preferred_element_type=jnp.float32)
    for r in range(1, nrel):
        out = out + lax.dot_general(acc_ref[r], w_ref[r],
                                    (((1,), (1,)), ((), ())),
                                    preferred_element_type=jnp.float32)
    out = jnp.maximum(out, 0.0)
    if normalize:
        nrm = jnp.sqrt(jnp.sum(out * out, axis=1, keepdims=True))
        out = out / jnp.maximum(nrm, 1e-12)
    o_ref[...] = out


def _transform(acc, w, normalize, bn):
    nrel, n, d = acc.shape
    grid = n // bn
    return pl.pallas_call(
        functools.partial(_transform_body, nrel, normalize),
        grid=(grid,),
        in_specs=[
            pl.BlockSpec((nrel, bn, d), lambda i: (0, i, 0)),
            pl.BlockSpec((nrel, d, d), lambda i: (0, 0, 0)),
        ],
        out_specs=pl.BlockSpec((bn, d), lambda i: (i, 0)),
        out_shape=jax.ShapeDtypeStruct((n, d), jnp.float32),
    )(acc, w)


def kernel(x, edge_index, edge_weight, ent_emb, rel_trans):
    n, f_in = x.shape
    d = ent_emb.shape[1]
    n_layers, nrel = rel_trans.shape[0], rel_trans.shape[1]
    e = edge_index.shape[2]

    rows_per_tile = -(-n // (NS * 8)) * 8
    n_pad = NS * rows_per_tile
    if n_pad != n:
        x = jnp.pad(x, ((0, n_pad - n), (0, 0)))

    cpt_h = -(-e // (NS * NH * CH))
    e_pad = NS * NH * cpt_h * CH
    dst = edge_index[:, 0, :]
    src = edge_index[:, 1, :]
    if e_pad != e:
        pad = ((0, 0), (0, e_pad - e))
        dst = jnp.pad(dst, pad)
        src = jnp.pad(src, pad)
        edge_weight = jnp.pad(edge_weight, pad)
    dst = dst.reshape(nrel, NS, NH, cpt_h, CH)
    src = src.reshape(nrel, NS, NH, cpt_h, CH)
    w = edge_weight.reshape(nrel, NS, NH, cpt_h, CH)

    bn = rows_per_tile
    mm = pl.pallas_call(
        _mm_body,
        grid=(n_pad // bn,),
        in_specs=[
            pl.BlockSpec((bn, f_in), lambda i: (i, 0)),
            pl.BlockSpec((f_in, d), lambda i: (0, 0)),
        ],
        out_specs=pl.BlockSpec((bn, d), lambda i: (i, 0)),
        out_shape=jax.ShapeDtypeStruct((n_pad, d), jnp.float32),
    )
    emb = mm(x, ent_emb)

    sc_agg = _make_sc_agg(n_pad, d, nrel, cpt_h)
    for l in range(n_layers):
        acc = sc_agg(emb, src, dst, w)
        emb = _transform(acc, rel_trans[l], normalize=(l == n_layers - 1),
                         bn=bn)
    return emb[:n]


# speedup vs baseline: 1.0684x; 1.0684x over previous
import functools

import jax
import jax.numpy as jnp
from jax import lax
from jax.experimental import pallas as pl
from jax.experimental.pallas import tpu as pltpu
from jax.experimental.pallas import tpu_sc as plsc

NC = 2
NS = 16
LANES = 16
CH = 128
NH = 2


def _sc_agg_body(nrel_per_core, cpt_h, rows_per_tile, d,
                 emb_hbm, src_hbm, dst_hbm, w_hbm, out_hbm,
                 src_v, dst_v, w_v, rows_a, rows_b, acc_sh,
                 sem_a, sem_b):
    c = lax.axis_index("c")
    s = lax.axis_index("s")
    base = s * rows_per_tile
    nd16 = d // LANES
    half = CH // 2
    zeros16 = jnp.zeros((LANES,), jnp.float32)

    def zero_row(i, _):
        for dd in range(nd16):
            rows_a[i, pl.ds(dd * LANES, LANES)] = zeros16
        return 0

    def scale_rows(rows, j):
        @plsc.parallel_loop(0, CH // LANES, unroll=2)
        def _(g):
            wv = rows_w(j, g)
            for ee in range(LANES):
                wb = wv[jnp.full((LANES,), ee, jnp.int32)]
                e = g * LANES + ee
                for dd in range(nd16):
                    sl = pl.ds(dd * LANES, LANES)
                    rows[e, sl] = rows[e, sl] * wb

    def rows_w(j, g):
        return w_v[j, pl.ds(g * LANES, LANES)]

    for rr in range(nrel_per_core):
        r = c * nrel_per_core + rr

        lax.fori_loop(0, CH, zero_row, 0)
        n_sub, rem = divmod(rows_per_tile, CH)
        for i in range(n_sub):
            pltpu.sync_copy(rows_a.at[pl.ds(0, CH)],
                            acc_sh.at[pl.ds(base + i * CH, CH)])
        if rem:
            pltpu.sync_copy(rows_a.at[pl.ds(0, rem)],
                            acc_sh.at[pl.ds(base + n_sub * CH, rem)])
        plsc.subcore_barrier()

        for h in range(NH):
            pltpu.sync_copy(src_hbm.at[r, s, h], src_v)
            pltpu.sync_copy(dst_hbm.at[r, s, h], dst_v)
            pltpu.sync_copy(w_hbm.at[r, s, h], w_v)

            pltpu.async_copy(emb_hbm.at[src_v.at[0]], rows_a, sem_a)

            def chunk_pair(k, _):
                j0 = 2 * k
                @pl.when(j0 + 1 < cpt_h)
                def _():
                    pltpu.async_copy(emb_hbm.at[src_v.at[j0 + 1]], rows_b,
                                     sem_b)
                pltpu.make_async_copy(emb_hbm.at[src_v.at[j0]], rows_a,
                                      sem_a).wait()
                scale_rows(rows_a, j0)
                pltpu.sync_copy(rows_a, acc_sh.at[dst_v.at[j0]], add=True)

                @pl.when(j0 + 1 < cpt_h)
                def _():
                    j1 = j0 + 1
                    @pl.when(j1 + 1 < cpt_h)
                    def _():
                        pltpu.async_copy(emb_hbm.at[src_v.at[j1 + 1]], rows_a,
                                         sem_a)
                    pltpu.make_async_copy(emb_hbm.at[src_v.at[j1]], rows_b,
                                          sem_b).wait()
                    scale_rows(rows_b, j1)
                    pltpu.sync_copy(rows_b, acc_sh.at[dst_v.at[j1]], add=True)
                return 0

            lax.fori_loop(0, (cpt_h + 1) // 2, chunk_pair, 0)

        plsc.subcore_barrier()
        pltpu.sync_copy(acc_sh.at[pl.ds(base, rows_per_tile)],
                        out_hbm.at[r, pl.ds(base, rows_per_tile)])
        plsc.subcore_barrier()


def _make_sc_agg(n, d, nrel, cpt_h):
    rows_per_tile = n // NS
    mesh = plsc.VectorSubcoreMesh(core_axis_name="c", subcore_axis_name="s")
    body = functools.partial(_sc_agg_body, nrel // NC, cpt_h, rows_per_tile, d)
    return pl.kernel(
        body,
        out_type=jax.ShapeDtypeStruct((nrel, n, d), jnp.float32),
        mesh=mesh,
        compiler_params=pltpu.CompilerParams(needs_layout_passes=False),
        scratch_types=[
            pltpu.VMEM((cpt_h, CH), jnp.int32),
            pltpu.VMEM((cpt_h, CH), jnp.int32),
            pltpu.VMEM((cpt_h, CH), jnp.float32),
            pltpu.VMEM((CH, d), jnp.float32),
            pltpu.VMEM((CH, d), jnp.float32),
            pltpu.VMEM_SHARED((n, d), jnp.float32),
            pltpu.SemaphoreType.DMA,
            pltpu.SemaphoreType.DMA,
        ],
    )


def _mm_body(x_ref, w_ref, o_ref):
    o_ref[...] = jnp.dot(x_ref[...], w_ref[...],
                         preferred_element_type=jnp.float32)


def _transform_body(nrel, normalize, acc_ref, w_ref, o_ref):
    out = lax.dot_general(acc_ref[0], w_ref[0], (((1,), (1,)), ((), ())),
                          preferred_element_type=jnp.float32)
    for r in range(1, nrel):
        out = out + lax.dot_general(acc_ref[r], w_ref[r],
                                    (((1,), (1,)), ((), ())),
                                    preferred_element_type=jnp.float32)
    out = jnp.maximum(out, 0.0)
    if normalize:
        nrm = jnp.sqrt(jnp.sum(out * out, axis=1, keepdims=True))
        out = out / jnp.maximum(nrm, 1e-12)
    o_ref[...] = out


def _transform(acc, w, normalize, bn):
    nrel, n, d = acc.shape
    grid = n // bn
    return pl.pallas_call(
        functools.partial(_transform_body, nrel, normalize),
        grid=(grid,),
        in_specs=[
            pl.BlockSpec((nrel, bn, d), lambda i: (0, i, 0)),
            pl.BlockSpec((nrel, d, d), lambda i: (0, 0, 0)),
        ],
        out_specs=pl.BlockSpec((bn, d), lambda i: (i, 0)),
        out_shape=jax.ShapeDtypeStruct((n, d), jnp.float32),
    )(acc, w)


def kernel(x, edge_index, edge_weight, ent_emb, rel_trans):
    n, f_in = x.shape
    d = ent_emb.shape[1]
    n_layers, nrel = rel_trans.shape[0], rel_trans.shape[1]
    e = edge_index.shape[2]

    rows_per_tile = -(-n // (NS * 8)) * 8
    n_pad = NS * rows_per_tile
    if n_pad != n:
        x = jnp.pad(x, ((0, n_pad - n), (0, 0)))

    cpt_h = -(-e // (NS * NH * CH))
    e_pad = NS * NH * cpt_h * CH
    dst = edge_index[:, 0, :]
    src = edge_index[:, 1, :]
    if e_pad != e:
        pad = ((0, 0), (0, e_pad - e))
        dst = jnp.pad(dst, pad)
        src = jnp.pad(src, pad)
        edge_weight = jnp.pad(edge_weight, pad)
    dst = dst.reshape(nrel, NS, NH, cpt_h, CH)
    src = src.reshape(nrel, NS, NH, cpt_h, CH)
    w = edge_weight.reshape(nrel, NS, NH, cpt_h, CH)

    bn = rows_per_tile
    mm = pl.pallas_call(
        _mm_body,
        grid=(n_pad // bn,),
        in_specs=[
            pl.BlockSpec((bn, f_in), lambda i: (i, 0)),
            pl.BlockSpec((f_in, d), lambda i: (0, 0)),
        ],
        out_specs=pl.BlockSpec((bn, d), lambda i: (i, 0)),
        out_shape=jax.ShapeDtypeStruct((n_pad, d), jnp.float32),
    )
    emb = mm(x, ent_emb)

    sc_agg = _make_sc_agg(n_pad, d, nrel, cpt_h)
    for l in range(n_layers):
        acc = sc_agg(emb, src, dst, w)
        emb = _transform(acc, rel_trans[l], normalize=(l == n_layers - 1),
                         bn=bn)
    return emb[:n]


# speedup vs baseline: 1.0689x; 1.0004x over previous
import functools

import jax
import jax.numpy as jnp
from jax import lax
from jax.experimental import pallas as pl
from jax.experimental.pallas import tpu as pltpu
from jax.experimental.pallas import tpu_sc as plsc

NC = 2
NS = 16
LANES = 16
CH = 128
NH = 2


def _sc_agg_body(nrel_per_core, cpt_h, rows_per_tile, d,
                 emb_hbm, src_hbm, dst_hbm, w_hbm, out_hbm,
                 src_v, dst_v, w_v, rows_a, rows_b, acc_sh,
                 sem_a, sem_b):
    c = lax.axis_index("c")
    s = lax.axis_index("s")
    base = s * rows_per_tile
    nd16 = d // LANES
    zeros16 = jnp.zeros((LANES,), jnp.float32)

    def zero_row(i, _):
        for dd in range(nd16):
            rows_a[i, pl.ds(dd * LANES, LANES)] = zeros16
        return 0

    def scale_rows(rows, j):
        @plsc.parallel_loop(0, CH // LANES, unroll=2)
        def _(g):
            wv = rows_w(j, g)
            for ee in range(LANES):
                wb = wv[jnp.full((LANES,), ee, jnp.int32)]
                e = g * LANES + ee
                for dd in range(nd16):
                    sl = pl.ds(dd * LANES, LANES)
                    rows[e, sl] = rows[e, sl] * wb

    def rows_w(j, g):
        return w_v[j, pl.ds(g * LANES, LANES)]

    for rr in range(nrel_per_core):
        r = c * nrel_per_core + rr

        lax.fori_loop(0, CH, zero_row, 0)
        n_sub, rem = divmod(rows_per_tile, CH)
        for i in range(n_sub):
            pltpu.sync_copy(rows_a.at[pl.ds(0, CH)],
                            acc_sh.at[pl.ds(base + i * CH, CH)])
        if rem:
            pltpu.sync_copy(rows_a.at[pl.ds(0, rem)],
                            acc_sh.at[pl.ds(base + n_sub * CH, rem)])
        plsc.subcore_barrier()

        for h in range(NH):
            pltpu.sync_copy(src_hbm.at[r, s, h], src_v)
            pltpu.sync_copy(dst_hbm.at[r, s, h], dst_v)
            pltpu.sync_copy(w_hbm.at[r, s, h], w_v)

            pltpu.async_copy(emb_hbm.at[src_v.at[0]], rows_a, sem_a)

            def chunk_pair(k, _):
                j0 = 2 * k
                @pl.when(j0 + 1 < cpt_h)
                def _():
                    pltpu.async_copy(emb_hbm.at[src_v.at[j0 + 1]], rows_b,
                                     sem_b)
                pltpu.make_async_copy(emb_hbm.at[src_v.at[j0]], rows_a,
                                      sem_a).wait()
                scale_rows(rows_a, j0)
                pltpu.sync_copy(rows_a, acc_sh.at[dst_v.at[j0]], add=True)

                @pl.when(j0 + 1 < cpt_h)
                def _():
                    j1 = j0 + 1
                    @pl.when(j1 + 1 < cpt_h)
                    def _():
                        pltpu.async_copy(emb_hbm.at[src_v.at[j1 + 1]], rows_a,
                                         sem_a)
                    pltpu.make_async_copy(emb_hbm.at[src_v.at[j1]], rows_b,
                                          sem_b).wait()
                    scale_rows(rows_b, j1)
                    pltpu.sync_copy(rows_b, acc_sh.at[dst_v.at[j1]], add=True)
                return 0

            lax.fori_loop(0, (cpt_h + 1) // 2, chunk_pair, 0)

        plsc.subcore_barrier()
        pltpu.sync_copy(acc_sh.at[pl.ds(base, rows_per_tile)],
                        out_hbm.at[r, pl.ds(base, rows_per_tile)])
        plsc.subcore_barrier()


def _make_sc_agg(n, d, nrel, cpt_h):
    rows_per_tile = n // NS
    mesh = plsc.VectorSubcoreMesh(core_axis_name="c", subcore_axis_name="s")
    body = functools.partial(_sc_agg_body, nrel // NC, cpt_h, rows_per_tile, d)
    return pl.kernel(
        body,
        out_type=jax.ShapeDtypeStruct((nrel, n, d), jnp.float32),
        mesh=mesh,
        compiler_params=pltpu.CompilerParams(needs_layout_passes=False),
        scratch_types=[
            pltpu.VMEM((cpt_h, CH), jnp.int32),
            pltpu.VMEM((cpt_h, CH), jnp.int32),
            pltpu.VMEM((cpt_h, CH), jnp.float32),
            pltpu.VMEM((CH, d), jnp.float32),
            pltpu.VMEM((CH, d), jnp.float32),
            pltpu.VMEM_SHARED((n, d), jnp.float32),
            pltpu.SemaphoreType.DMA,
            pltpu.SemaphoreType.DMA,
        ],
    )


def _mm_body(x_ref, w_ref, o_ref):
    o_ref[...] = jnp.dot(x_ref[...], w_ref[...],
                         preferred_element_type=jnp.float32)


def _transform_body(nrel, normalize, acc_ref, w_ref, o_ref):
    out = lax.dot_general(acc_ref[0], w_ref[0], (((1,), (1,)), ((), ())),
                          preferred_element_type=jnp.float32)
    for r in range(1, nrel):
        out = out + lax.dot_general(acc_ref[r], w_ref[r],
                                    (((1,), (1,)), ((), ())),
                                    preferred_element_type=jnp.float32)
    out = jnp.maximum(out, 0.0)
    if normalize:
        nrm = jnp.sqrt(jnp.sum(out * out, axis=1, keepdims=True))
        out = out / jnp.maximum(nrm, 1e-12)
    o_ref[...] = out


def _transform(acc, w, normalize, bn):
    nrel, n, d = acc.shape
    grid = n // bn
    return pl.pallas_call(
        functools.partial(_transform_body, nrel, normalize),
        grid=(grid,),
        in_specs=[
            pl.BlockSpec((nrel, bn, d), lambda i: (0, i, 0)),
            pl.BlockSpec((nrel, d, d), lambda i: (0, 0, 0)),
        ],
        out_specs=pl.BlockSpec((bn, d), lambda i: (i, 0)),
        out_shape=jax.ShapeDtypeStruct((n, d), jnp.float32),
    )(acc, w)


def kernel(x, edge_index, edge_weight, ent_emb, rel_trans):
    n, f_in = x.shape
    d = ent_emb.shape[1]
    n_layers, nrel = rel_trans.shape[0], rel_trans.shape[1]
    e = edge_index.shape[2]

    rows_per_tile = -(-n // (NS * 8)) * 8
    n_pad = NS * rows_per_tile
    if n_pad != n:
        x = jnp.pad(x, ((0, n_pad - n), (0, 0)))

    cpt_h = -(-e // (NS * NH * CH))
    e_pad = NS * NH * cpt_h * CH
    dst = edge_index[:, 0, :]
    src = edge_index[:, 1, :]
    if e_pad != e:
        pad = ((0, 0), (0, e_pad - e))
        dst = jnp.pad(dst, pad)
        src = jnp.pad(src, pad)
        edge_weight = jnp.pad(edge_weight, pad)
    dst = dst.reshape(nrel, NS, NH, cpt_h, CH)
    src = src.reshape(nrel, NS, NH, cpt_h, CH)
    w = edge_weight.reshape(nrel, NS, NH, cpt_h, CH)

    bn = rows_per_tile
    mm = pl.pallas_call(
        _mm_body,
        grid=(n_pad // bn,),
        in_specs=[
            pl.BlockSpec((bn, f_in), lambda i: (i, 0)),
            pl.BlockSpec((f_in, d), lambda i: (0, 0)),
        ],
        out_specs=pl.BlockSpec((bn, d), lambda i: (i, 0)),
        out_shape=jax.ShapeDtypeStruct((n_pad, d), jnp.float32),
    )
    emb = mm(x, ent_emb)

    sc_agg = _make_sc_agg(n_pad, d, nrel, cpt_h)
    for l in range(n_layers):
        acc = sc_agg(emb, src, dst, w)
        emb = _transform(acc, rel_trans[l], normalize=(l == n_layers - 1),
                         bn=bn)
    return emb[:n]
